# baseline (speedup 1.0000x reference)
import functools

import jax
import jax.numpy as jnp
from jax import lax
from jax.experimental import pallas as pl
from jax.experimental.pallas import tpu as pltpu
from jax.experimental.pallas import tpu_sc as plsc

B = 16
N_ITEMS = 8192
TOTAL = 4096
D_H = 512
D_Z = 256
TAU = 0.2


_NW = 32
_CH = 4
_CHP = 8


def _sc_gather(item_idx, gram):
    n_rows = item_idx.shape[0]
    rpw = n_rows // _NW
    nch = rpw // _CH
    idx_pad = jnp.pad(
        item_idx.reshape(_NW, nch, _CH), ((0, 0), (0, 0), (0, _CHP - _CH))
    ).reshape(-1)
    mesh = plsc.VectorSubcoreMesh(core_axis_name="c", subcore_axis_name="s")

    @functools.partial(
        pl.kernel,
        out_type=jax.ShapeDtypeStruct((n_rows, N_ITEMS), jnp.float32),
        mesh=mesh,
        scratch_types=[
            pltpu.VMEM((nch * _CHP,), jnp.int32),
            pltpu.VMEM((_CH, N_ITEMS), jnp.float32),
            pltpu.VMEM((_CH, N_ITEMS), jnp.float32),
            pltpu.SemaphoreType.DMA,
            pltpu.SemaphoreType.DMA,
        ],
    )
    def gk(idx_hbm, gram_hbm, out_hbm, idx_v, buf0, buf1, sem0, sem1):
        wid = lax.axis_index("s") * 2 + lax.axis_index("c")
        base = wid * rpw
        bufs = (buf0, buf1)
        sems = (sem0, sem1)
        pltpu.sync_copy(idx_hbm.at[pl.ds(wid * nch * _CHP, nch * _CHP)],
                        idx_v)
        pend = [None, None]
        pend[0] = pltpu.async_copy(
            gram_hbm.at[idx_v.at[pl.ds(0, _CH)]], bufs[0], sems[0])
        for c in range(nch):
            n = c + 1
            if n < nch:
                pend[n % 2] = pltpu.async_copy(
                    gram_hbm.at[idx_v.at[pl.ds(n * _CHP, _CH)]],
                    bufs[n % 2], sems[n % 2])
            pend[c % 2].wait()
            pltpu.sync_copy(bufs[c % 2],
                            out_hbm.at[pl.ds(base + c * _CH, _CH)])

    return gk(idx_pad, gram)


_T = 512
_K = 2048
_LN2 = 0.6931471805599453


def _tc_body(sims_ref, rat_ref, oh_ref, w1_ref, w2_ref, b1_ref, b2_ref,
             items_ref, z_ref, out_ref, kl_ref,
             acc_ref, ss_ref, klacc_ref, bin_ref, cnt_ref, itn_ref):
    t = pl.program_id(0)
    k = pl.program_id(1)
    nt = pl.num_programs(0)
    nk = pl.num_programs(1)

    @pl.when(k == 0)
    def _():
        acc_ref[...] = jnp.zeros_like(acc_ref)
        ss_ref[...] = jnp.zeros_like(ss_ref)

    @pl.when(jnp.logical_and(k == 0, t == 0))
    def _():
        klacc_ref[...] = jnp.zeros_like(klacc_ref)
        out_ref[...] = jnp.zeros_like(out_ref)
        cnt_ref[...] = jnp.zeros_like(cnt_ref)
        it = items_ref[...]
        itn = it / jnp.maximum(
            jnp.sqrt(jnp.sum(it * it, axis=1, keepdims=True)), 1e-12)
        itn_ref[...] = itn.astype(jnp.bfloat16)

    @pl.when(t == 0)
    def _():
        bin_ref[:, pl.ds(k * _K, _K)] = jnp.where(
            rat_ref[...] > 0.0, 1.0, 0.0).astype(jnp.bfloat16)

    oh = oh_ref[...]
    bin_bf = bin_ref[:, pl.ds(k * _K, _K)]
    mask = jnp.dot(oh.astype(jnp.bfloat16), bin_bf,
                   preferred_element_type=jnp.float32)
    sims_bf = sims_ref[...].astype(jnp.bfloat16)
    inp = sims_bf * mask.astype(jnp.bfloat16)
    sq = sims_bf * sims_bf
    s2 = lax.dot_general(sq, bin_bf, (((1,), (1,)), ((), ())),
                         preferred_element_type=jnp.float32)
    ss_ref[...] += jnp.sum(s2 * oh, axis=1, keepdims=True)
    acc_ref[...] += jnp.dot(inp, w1_ref[...],
                            preferred_element_type=jnp.float32)

    @pl.when(k == nk - 1)
    def _():
        scale = 1.0 / jnp.maximum(jnp.sqrt(ss_ref[...]), 1e-12)
        h = jnp.tanh(acc_ref[...] * scale + b1_ref[...])
        x = jnp.dot(h.astype(jnp.bfloat16), w2_ref[...],
                    preferred_element_type=jnp.float32) + b2_ref[...]
        mean_ = x[:, :D_Z]
        logvar = x[:, D_Z:]
        z_ref[...] = mean_
        klacc_ref[...] += jnp.sum(
            mean_ * mean_ + jnp.exp(logvar) - 1.0 - logvar)

        zn = mean_ * ((1.0 / (TAU * _LN2)) / jnp.maximum(
            jnp.sqrt(jnp.sum(mean_ * mean_, axis=1, keepdims=True)), 1e-12))
        sim2 = lax.dot_general(zn.astype(jnp.bfloat16), itn_ref[...],
                               (((1,), (1,)), ((), ())),
                               preferred_element_type=jnp.float32)
        e = jnp.exp2(sim2)
        out_ref[...] += lax.dot_general(oh, e, (((0,), (0,)), ((), ())),
                                        preferred_element_type=jnp.float32)
        cnt_ref[...] += jnp.sum(oh, axis=0)[:, None]

        @pl.when(t == nt - 1)
        def _():
            kl_ref[...] = klacc_ref[...]
            cnt = cnt_ref[...]
            mean_out = out_ref[...] / jnp.maximum(cnt, 1.0)
            out_ref[...] = jnp.where(cnt > 0.0, jnp.log(mean_out + 1.0), 0.0)


def _tc_fused(sims, rating, onehot, W1, W2, b1, b2, items):
    n_tok = sims.shape[0]
    grid = (n_tok // _T, N_ITEMS // _K)
    return pl.pallas_call(
        _tc_body,
        grid=grid,
        in_specs=[
            pl.BlockSpec((_T, _K), lambda t, k: (t, k)),
            pl.BlockSpec((B, _K), lambda t, k: (0, k)),
            pl.BlockSpec((_T, B), lambda t, k: (t, 0)),
            pl.BlockSpec((_K, D_H), lambda t, k: (k, 0)),
            pl.BlockSpec((D_H, 2 * D_Z), lambda t, k: (0, 0)),
            pl.BlockSpec((1, D_H), lambda t, k: (0, 0)),
            pl.BlockSpec((1, 2 * D_Z), lambda t, k: (0, 0)),
            pl.BlockSpec((N_ITEMS, D_Z), lambda t, k: (0, 0)),
        ],
        out_specs=[
            pl.BlockSpec((_T, D_Z), lambda t, k: (t, 0)),
            pl.BlockSpec((B, N_ITEMS), lambda t, k: (0, 0)),
            pl.BlockSpec((1, 1), lambda t, k: (0, 0)),
        ],
        out_shape=[
            jax.ShapeDtypeStruct((n_tok, D_Z), jnp.float32),
            jax.ShapeDtypeStruct((B, N_ITEMS), jnp.float32),
            jax.ShapeDtypeStruct((1, 1), jnp.float32),
        ],
        scratch_shapes=[
            pltpu.VMEM((_T, D_H), jnp.float32),
            pltpu.VMEM((_T, 1), jnp.float32),
            pltpu.VMEM((1, 1), jnp.float32),
            pltpu.VMEM((B, N_ITEMS), jnp.bfloat16),
            pltpu.VMEM((B, 1), jnp.float32),
            pltpu.VMEM((N_ITEMS, D_Z), jnp.bfloat16),
        ],
        compiler_params=pltpu.CompilerParams(
            dimension_semantics=("arbitrary", "arbitrary")),
    )(sims, rating, onehot, W1, W2, b1, b2, items)


def kernel(rating_matrix_batch, item_idx, segment_ids, gram, W1, b1, W2, b2,
           items):
    onehot = (segment_ids[:, None] == jnp.arange(B, dtype=jnp.int32)[None, :]
              ).astype(jnp.float32)
    sims = _sc_gather(item_idx, gram)
    z, new_output, kl = _tc_fused(
        sims, rating_matrix_batch, onehot,
        W1.astype(jnp.bfloat16), W2.astype(jnp.bfloat16),
        b1.reshape(1, D_H), b2.reshape(1, 2 * D_Z), items)
    return (z, new_output, kl[0, 0] * (0.5 / TOTAL))


# speedup vs baseline: 1.0304x; 1.0304x over previous
import functools

import jax
import jax.numpy as jnp
from jax import lax
from jax.experimental import pallas as pl
from jax.experimental.pallas import tpu as pltpu
from jax.experimental.pallas import tpu_sc as plsc

B = 16
N_ITEMS = 8192
TOTAL = 4096
D_H = 512
D_Z = 256
TAU = 0.2


_NW = 32
_CH = 4
_CHP = 8


def _sc_gather(item_idx, gram):
    n_rows = item_idx.shape[0]
    rpw = n_rows // _NW
    nch = rpw // _CH
    idx_pad = jnp.pad(
        item_idx.reshape(_NW, nch, _CH), ((0, 0), (0, 0), (0, _CHP - _CH))
    ).reshape(-1)
    mesh = plsc.VectorSubcoreMesh(core_axis_name="c", subcore_axis_name="s")

    @functools.partial(
        pl.kernel,
        out_type=jax.ShapeDtypeStruct((n_rows, N_ITEMS), jnp.float32),
        mesh=mesh,
        scratch_types=[
            pltpu.VMEM((nch * _CHP,), jnp.int32),
            pltpu.VMEM((_CH, N_ITEMS), jnp.float32),
            pltpu.VMEM((_CH, N_ITEMS), jnp.float32),
            pltpu.SemaphoreType.DMA,
            pltpu.SemaphoreType.DMA,
        ],
    )
    def gk(idx_hbm, gram_hbm, out_hbm, idx_v, buf0, buf1, sem0, sem1):
        wid = lax.axis_index("s") * 2 + lax.axis_index("c")
        base = wid * rpw
        bufs = (buf0, buf1)
        sems = (sem0, sem1)
        pltpu.sync_copy(idx_hbm.at[pl.ds(wid * nch * _CHP, nch * _CHP)],
                        idx_v)
        pend = [None, None]
        pend[0] = pltpu.async_copy(
            gram_hbm.at[idx_v.at[pl.ds(0, _CH)]], bufs[0], sems[0])
        for c in range(nch):
            n = c + 1
            if n < nch:
                pend[n % 2] = pltpu.async_copy(
                    gram_hbm.at[idx_v.at[pl.ds(n * _CHP, _CH)]],
                    bufs[n % 2], sems[n % 2])
            pend[c % 2].wait()
            pltpu.sync_copy(bufs[c % 2],
                            out_hbm.at[pl.ds(base + c * _CH, _CH)])

    return gk(idx_pad, gram)


_T = 1024
_K = 2048


def _enc_body(sims_ref, rat_ref, oh_ref, w1_ref, w2_ref, b1_ref, b2_ref,
              z_ref, kl_ref, acc_ref, ss_ref, klacc_ref, bin_ref):
    t = pl.program_id(0)
    k = pl.program_id(1)
    nt = pl.num_programs(0)
    nk = pl.num_programs(1)

    @pl.when(k == 0)
    def _():
        acc_ref[...] = jnp.zeros_like(acc_ref)
        ss_ref[...] = jnp.zeros_like(ss_ref)

    @pl.when(jnp.logical_and(k == 0, t == 0))
    def _():
        klacc_ref[...] = jnp.zeros_like(klacc_ref)

    @pl.when(t == 0)
    def _():
        bin_ref[:, pl.ds(k * _K, _K)] = jnp.where(
            rat_ref[...] > 0.0, 1.0, 0.0).astype(jnp.bfloat16)

    oh = oh_ref[...]
    bin_bf = bin_ref[:, pl.ds(k * _K, _K)]
    mask = jnp.dot(oh.astype(jnp.bfloat16), bin_bf,
                   preferred_element_type=jnp.float32)
    sims_bf = sims_ref[...].astype(jnp.bfloat16)
    inp = sims_bf * mask.astype(jnp.bfloat16)
    sq = sims_bf * sims_bf
    s2 = lax.dot_general(sq, bin_bf, (((1,), (1,)), ((), ())),
                         preferred_element_type=jnp.float32)
    ss_ref[...] += jnp.sum(s2 * oh, axis=1, keepdims=True)
    acc_ref[...] += jnp.dot(inp, w1_ref[...],
                            preferred_element_type=jnp.float32)

    @pl.when(k == nk - 1)
    def _():
        scale = 1.0 / jnp.maximum(jnp.sqrt(ss_ref[...]), 1e-12)
        h = jnp.tanh(acc_ref[...] * scale + b1_ref[...])
        x = jnp.dot(h.astype(jnp.bfloat16), w2_ref[...],
                    preferred_element_type=jnp.float32) + b2_ref[...]
        mean_ = x[:, :D_Z]
        logvar = x[:, D_Z:]
        z_ref[...] = mean_
        klacc_ref[...] += jnp.sum(
            mean_ * mean_ + jnp.exp(logvar) - 1.0 - logvar)

        @pl.when(t == nt - 1)
        def _():
            kl_ref[...] = klacc_ref[...]


def _encoder(sims, rating, onehot, W1, W2, b1, b2):
    n_tok = sims.shape[0]
    grid = (n_tok // _T, N_ITEMS // _K)
    return pl.pallas_call(
        _enc_body,
        grid=grid,
        in_specs=[
            pl.BlockSpec((_T, _K), lambda t, k: (t, k)),
            pl.BlockSpec((B, _K), lambda t, k: (0, k)),
            pl.BlockSpec((_T, B), lambda t, k: (t, 0)),
            pl.BlockSpec((_K, D_H), lambda t, k: (k, 0)),
            pl.BlockSpec((D_H, 2 * D_Z), lambda t, k: (0, 0)),
            pl.BlockSpec((1, D_H), lambda t, k: (0, 0)),
            pl.BlockSpec((1, 2 * D_Z), lambda t, k: (0, 0)),
        ],
        out_specs=[
            pl.BlockSpec((_T, D_Z), lambda t, k: (t, 0)),
            pl.BlockSpec((1, 1), lambda t, k: (0, 0)),
        ],
        out_shape=[
            jax.ShapeDtypeStruct((n_tok, D_Z), jnp.float32),
            jax.ShapeDtypeStruct((1, 1), jnp.float32),
        ],
        scratch_shapes=[
            pltpu.VMEM((_T, D_H), jnp.float32),
            pltpu.VMEM((_T, 1), jnp.float32),
            pltpu.VMEM((1, 1), jnp.float32),
            pltpu.VMEM((B, N_ITEMS), jnp.bfloat16),
        ],
        compiler_params=pltpu.CompilerParams(
            dimension_semantics=("arbitrary", "arbitrary")),
    )(sims, rating, onehot, W1, W2, b1, b2)


_T2 = 1024
_LN2 = 0.6931471805599453


def _dec_body(z_ref, oh_ref, items_ref, out_ref, cnt_ref, itn_ref):
    t = pl.program_id(0)
    nt = pl.num_programs(0)

    @pl.when(t == 0)
    def _():
        out_ref[...] = jnp.zeros_like(out_ref)
        cnt_ref[...] = jnp.zeros_like(cnt_ref)
        it = items_ref[...]
        itn = it / jnp.maximum(
            jnp.sqrt(jnp.sum(it * it, axis=1, keepdims=True)), 1e-12)
        itn_ref[...] = itn.astype(jnp.bfloat16)

    z = z_ref[...]
    zn = z * ((1.0 / (TAU * _LN2)) / jnp.maximum(
        jnp.sqrt(jnp.sum(z * z, axis=1, keepdims=True)), 1e-12))
    sim2 = lax.dot_general(zn.astype(jnp.bfloat16), itn_ref[...],
                           (((1,), (1,)), ((), ())),
                           preferred_element_type=jnp.float32)
    e = jnp.exp2(sim2)
    oh = oh_ref[...]
    out_ref[...] += lax.dot_general(oh, e, (((0,), (0,)), ((), ())),
                                    preferred_element_type=jnp.float32)
    cnt_ref[...] += jnp.sum(oh, axis=0)[:, None]

    @pl.when(t == nt - 1)
    def _():
        cnt = cnt_ref[...]
        mean_out = out_ref[...] / jnp.maximum(cnt, 1.0)
        out_ref[...] = jnp.where(cnt > 0.0, jnp.log(mean_out + 1.0), 0.0)


def _decoder(z, onehot, items):
    grid = (TOTAL // _T2,)
    return pl.pallas_call(
        _dec_body,
        grid=grid,
        in_specs=[
            pl.BlockSpec((_T2, D_Z), lambda t: (t, 0)),
            pl.BlockSpec((_T2, B), lambda t: (t, 0)),
            pl.BlockSpec((N_ITEMS, D_Z), lambda t: (0, 0)),
        ],
        out_specs=pl.BlockSpec((B, N_ITEMS), lambda t: (0, 0)),
        out_shape=jax.ShapeDtypeStruct((B, N_ITEMS), jnp.float32),
        scratch_shapes=[
            pltpu.VMEM((B, 1), jnp.float32),
            pltpu.VMEM((N_ITEMS, D_Z), jnp.bfloat16),
        ],
        compiler_params=pltpu.CompilerParams(
            dimension_semantics=("arbitrary",)),
    )(z, onehot, items)


def kernel(rating_matrix_batch, item_idx, segment_ids, gram, W1, b1, W2, b2,
           items):
    onehot = (segment_ids[:, None] == jnp.arange(B, dtype=jnp.int32)[None, :]
              ).astype(jnp.float32)
    sims = _sc_gather(item_idx, gram)
    z, kl = _encoder(sims, rating_matrix_batch, onehot,
                     W1.astype(jnp.bfloat16), W2.astype(jnp.bfloat16),
                     b1.reshape(1, D_H), b2.reshape(1, 2 * D_Z))
    new_output = _decoder(z, onehot, items)
    return (z, new_output, kl[0, 0] * (0.5 / TOTAL))
